# Initial kernel scaffold; baseline (speedup 1.0000x reference)
import functools

import jax
import jax.numpy as jnp
from jax import lax
from jax.experimental import pallas as pl
from jax.experimental.pallas import tpu as pltpu
from jax.experimental.pallas import tpu_sc as plsc

N = 50000
E = 800000
WIDTH = 64
HALF = 32
DEPTH = 4

LANES = 128
N_PAD = 50176
EROWS = 6272
E_PAD = EROWS * LANES
RPT = N_PAD // 16
ERPT = EROWS // 16
ERPC = EROWS // 2
ERPT_DEG = ERPC // 16

_mesh = plsc.VectorSubcoreMesh(core_axis_name="c", subcore_axis_name="s")


@functools.partial(
    pl.kernel,
    out_type=jax.ShapeDtypeStruct((2, N_PAD, 8), jnp.float32),
    mesh=_mesh,
    scratch_types=[
        pltpu.VMEM((ERPT_DEG, LANES), jnp.int32),
        pltpu.VMEM((LANES, 8), jnp.float32),
        pltpu.VMEM_SHARED((N_PAD, 8), jnp.float32),
        pltpu.SemaphoreType.DMA,
    ],
)
def _sc_degree(dst2d, zeros8, ones8, degp, dstv, onesv, deg_sh, sem):
    cid = lax.axis_index("c")
    sid = lax.axis_index("s")
    pltpu.sync_copy(zeros8.at[pl.ds(sid * RPT, RPT)],
                    deg_sh.at[pl.ds(sid * RPT, RPT)])
    pltpu.sync_copy(ones8, onesv)

    def half(c):
        base = c * ERPC + sid * ERPT_DEG
        pltpu.sync_copy(dst2d.at[pl.ds(base, ERPT_DEG)], dstv)
        plsc.subcore_barrier()

        def step(j, carry):
            pltpu.sync_copy(onesv, deg_sh.at[dstv.at[j]], add=True)
            return carry

        lax.fori_loop(0, ERPT_DEG, step, 0)
        plsc.subcore_barrier()
        pltpu.sync_copy(deg_sh.at[pl.ds(sid * RPT, RPT)],
                        degp.at[c, pl.ds(sid * RPT, RPT)])

    pl.when(cid == 0)(lambda: half(0))
    pl.when(cid == 1)(lambda: half(1))


@functools.partial(
    pl.kernel,
    out_type=[jax.ShapeDtypeStruct((N_PAD, HALF), jnp.float32),
              jax.ShapeDtypeStruct((N_PAD, HALF), jnp.float32)],
    mesh=_mesh,
    scratch_types=[
        pltpu.VMEM((ERPT, LANES), jnp.int32),
        pltpu.VMEM((ERPT, LANES), jnp.int32),
        pltpu.VMEM((LANES, HALF), jnp.float32),
        pltpu.VMEM_SHARED((N_PAD, HALF), jnp.float32),
        pltpu.SemaphoreType.DMA,
    ],
)
def _sc_scatter(y0, y1, src2d, dst2d, z0, z1, srcv, dstv, rows, z_sh, sem):
    cid = lax.axis_index("c")
    sid = lax.axis_index("s")
    pltpu.sync_copy(src2d.at[pl.ds(sid * ERPT, ERPT)], srcv)
    pltpu.sync_copy(dst2d.at[pl.ds(sid * ERPT, ERPT)], dstv)

    def half(y_hbm, z_hbm):
        pltpu.sync_copy(y_hbm.at[pl.ds(sid * RPT, RPT)],
                        z_sh.at[pl.ds(sid * RPT, RPT)])
        plsc.subcore_barrier()

        def step(j, carry):
            pltpu.async_copy(y_hbm.at[srcv.at[j]], rows, sem).wait()
            pltpu.sync_copy(rows, z_sh.at[dstv.at[j]], add=True)
            return carry

        lax.fori_loop(0, ERPT, step, 0)
        plsc.subcore_barrier()
        pltpu.sync_copy(z_sh.at[pl.ds(sid * RPT, RPT)],
                        z_hbm.at[pl.ds(sid * RPT, RPT)])

    pl.when(cid == 0)(lambda: half(y0, z0))
    pl.when(cid == 1)(lambda: half(y1, z1))


BN = 1024
GRID = N_PAD // BN


def _tc_pre_body(x, degp, fc1_W, fc1_b, conv_W, y0, y1, dinv):
    deg = degp[0, :, 0:1] + degp[1, :, 0:1] + 1.0
    di = lax.rsqrt(deg)
    h = jnp.dot(x[...], fc1_W[...], preferred_element_type=jnp.float32)
    h = h + fc1_b[...]
    y = jnp.dot(h, conv_W[...], preferred_element_type=jnp.float32) * di
    y0[...] = y[:, :HALF]
    y1[...] = y[:, HALF:]
    dinv[...] = di


_tc_pre = pl.pallas_call(
    _tc_pre_body,
    grid=(GRID,),
    in_specs=[
        pl.BlockSpec((BN, 3), lambda i: (i, 0)),
        pl.BlockSpec((2, BN, 8), lambda i: (0, i, 0)),
        pl.BlockSpec((3, WIDTH), lambda i: (0, 0)),
        pl.BlockSpec((1, WIDTH), lambda i: (0, 0)),
        pl.BlockSpec((WIDTH, WIDTH), lambda i: (0, 0)),
    ],
    out_specs=[
        pl.BlockSpec((BN, HALF), lambda i: (i, 0)),
        pl.BlockSpec((BN, HALF), lambda i: (i, 0)),
        pl.BlockSpec((BN, 1), lambda i: (i, 0)),
    ],
    out_shape=[
        jax.ShapeDtypeStruct((N_PAD, HALF), jnp.float32),
        jax.ShapeDtypeStruct((N_PAD, HALF), jnp.float32),
        jax.ShapeDtypeStruct((N_PAD, 1), jnp.float32),
    ],
)


def _tc_mid_body(z0, z1, dinv, conv_W, conv_b, y0, y1):
    di = dinv[...]
    z = jnp.concatenate([z0[...], z1[...]], axis=1)
    h = jnp.maximum(z * di + conv_b[...], 0.0)
    y = jnp.dot(h, conv_W[...], preferred_element_type=jnp.float32) * di
    y0[...] = y[:, :HALF]
    y1[...] = y[:, HALF:]


_tc_mid = pl.pallas_call(
    _tc_mid_body,
    grid=(GRID,),
    in_specs=[
        pl.BlockSpec((BN, HALF), lambda i: (i, 0)),
        pl.BlockSpec((BN, HALF), lambda i: (i, 0)),
        pl.BlockSpec((BN, 1), lambda i: (i, 0)),
        pl.BlockSpec((WIDTH, WIDTH), lambda i: (0, 0)),
        pl.BlockSpec((1, WIDTH), lambda i: (0, 0)),
    ],
    out_specs=[
        pl.BlockSpec((BN, HALF), lambda i: (i, 0)),
        pl.BlockSpec((BN, HALF), lambda i: (i, 0)),
    ],
    out_shape=[
        jax.ShapeDtypeStruct((N_PAD, HALF), jnp.float32),
        jax.ShapeDtypeStruct((N_PAD, HALF), jnp.float32),
    ],
)


def _tc_post_body(z0, z1, dinv, conv_b, fc2_W, fc2_b, out):
    di = dinv[...]
    z = jnp.concatenate([z0[...], z1[...]], axis=1)
    h = jnp.maximum(z * di + conv_b[...], 0.0)
    out[...] = jnp.dot(h, fc2_W[...], preferred_element_type=jnp.float32) + fc2_b[...]


_tc_post = pl.pallas_call(
    _tc_post_body,
    grid=(GRID,),
    in_specs=[
        pl.BlockSpec((BN, HALF), lambda i: (i, 0)),
        pl.BlockSpec((BN, HALF), lambda i: (i, 0)),
        pl.BlockSpec((BN, 1), lambda i: (i, 0)),
        pl.BlockSpec((1, WIDTH), lambda i: (0, 0)),
        pl.BlockSpec((WIDTH, 1), lambda i: (0, 0)),
        pl.BlockSpec((1, 1), lambda i: (0, 0)),
    ],
    out_specs=pl.BlockSpec((BN, 1), lambda i: (i, 0)),
    out_shape=jax.ShapeDtypeStruct((N_PAD, 1), jnp.float32),
)


def kernel(x, edge_index, fc1_W, fc1_b, conv_W, conv_b, fc2_W, fc2_b):
    src = jnp.concatenate(
        [edge_index[0], jnp.zeros((E_PAD - E,), jnp.int32)]).reshape(EROWS, LANES)
    dst = jnp.concatenate(
        [edge_index[1], jnp.full((E_PAD - E,), N, jnp.int32)]).reshape(EROWS, LANES)
    x_pad = jnp.concatenate([x, jnp.zeros((N_PAD - N, 3), x.dtype)], axis=0)
    zeros8 = jnp.zeros((N_PAD, 8), jnp.float32)
    ones8 = jnp.ones((LANES, 8), jnp.float32)

    degp = _sc_degree(dst, zeros8, ones8)

    y0, y1, dinv = _tc_pre(x_pad, degp, fc1_W, fc1_b.reshape(1, WIDTH), conv_W)

    conv_b2 = conv_b.reshape(1, WIDTH)
    for layer in range(DEPTH):
        z0, z1 = _sc_scatter(y0, y1, src, dst)
        if layer < DEPTH - 1:
            y0, y1 = _tc_mid(z0, z1, dinv, conv_W, conv_b2)
        else:
            out = _tc_post(z0, z1, dinv, conv_b2,
                           fc2_W, fc2_b.reshape(1, 1))
    return out[:N]


# speedup vs baseline: 3.6013x; 3.6013x over previous
import functools

import jax
import jax.numpy as jnp
from jax import lax
from jax.experimental import pallas as pl
from jax.experimental.pallas import tpu as pltpu
from jax.experimental.pallas import tpu_sc as plsc

N = 50000
E = 800000
WIDTH = 64
NQ = 8
QW = WIDTH // NQ
DEPTH = 4

LANES = 128
N_PAD = 50176
EROWS = 6400
E_PAD = EROWS * LANES
RPT = N_PAD // 16
ERPT = EROWS // 16


def _sc_scatter_body(*refs):
    ys = refs[:NQ]
    src2d, dst2d = refs[NQ], refs[NQ + 1]
    zs = refs[NQ + 2:2 * NQ + 2]
    srcv, dstv, rows, z_sh, sem = refs[2 * NQ + 2:]
    cid = lax.axis_index("c")
    sid = lax.axis_index("s")
    pltpu.sync_copy(src2d.at[pl.ds(sid * ERPT, ERPT)], srcv)
    pltpu.sync_copy(dst2d.at[pl.ds(sid * ERPT, ERPT)], dstv)

    def one_slice(y_hbm, z_hbm):
        pltpu.sync_copy(y_hbm.at[pl.ds(sid * RPT, RPT)],
                        z_sh.at[pl.ds(sid * RPT, RPT)])
        plsc.subcore_barrier()

        def step(j, carry):
            pltpu.async_copy(y_hbm.at[srcv.at[j]], rows, sem).wait()
            pltpu.sync_copy(rows, z_sh.at[dstv.at[j]], add=True)
            return carry

        lax.fori_loop(0, ERPT, step, 0)
        plsc.subcore_barrier()
        pltpu.sync_copy(z_sh.at[pl.ds(sid * RPT, RPT)],
                        z_hbm.at[pl.ds(sid * RPT, RPT)])

    def core(c):
        for q in range(c, NQ, 2):
            one_slice(ys[q], zs[q])

    pl.when(cid == 0)(lambda: core(0))
    pl.when(cid == 1)(lambda: core(1))


@functools.cache
def _sc_kernels():
    mesh = plsc.VectorSubcoreMesh(core_axis_name="c", subcore_axis_name="s")
    params = pltpu.CompilerParams(use_tc_tiling_on_sc=False)
    qshape = jax.ShapeDtypeStruct((N_PAD, QW), jnp.float32)
    scatter = pl.kernel(
        _sc_scatter_body,
        out_type=[qshape] * NQ,
        mesh=mesh,
        scratch_types=[
            pltpu.VMEM((ERPT, LANES), jnp.int32),
            pltpu.VMEM((ERPT, LANES), jnp.int32),
            pltpu.VMEM((LANES, QW), jnp.float32),
            pltpu.VMEM_SHARED((N_PAD, QW), jnp.float32),
            pltpu.SemaphoreType.DMA,
        ],
        compiler_params=params,
    )
    return scatter


BN = 1024
GRID = N_PAD // BN


def _split(y, outs):
    for q, ref in enumerate(outs):
        ref[...] = y[:, q * QW:(q + 1) * QW]


def _tc_pre_body(x, degz, fc1_W, fc1_b, conv_W, *outs):
    di = lax.rsqrt(degz[:, 0:1])
    h = jnp.dot(x[...], fc1_W[...], preferred_element_type=jnp.float32)
    h = h + fc1_b[...]
    y = jnp.dot(h, conv_W[...], preferred_element_type=jnp.float32) * di
    _split(y, outs[:NQ])
    outs[NQ][...] = di


_qspec = pl.BlockSpec((BN, QW), lambda i: (i, 0))
_qshape = jax.ShapeDtypeStruct((N_PAD, QW), jnp.float32)
_dspec = pl.BlockSpec((BN, 1), lambda i: (i, 0))

_tc_pre = pl.pallas_call(
    _tc_pre_body,
    grid=(GRID,),
    in_specs=[
        pl.BlockSpec((BN, 3), lambda i: (i, 0)),
        _qspec,
        pl.BlockSpec((3, WIDTH), lambda i: (0, 0)),
        pl.BlockSpec((1, WIDTH), lambda i: (0, 0)),
        pl.BlockSpec((WIDTH, WIDTH), lambda i: (0, 0)),
    ],
    out_specs=[_qspec] * NQ + [_dspec],
    out_shape=[_qshape] * NQ + [jax.ShapeDtypeStruct((N_PAD, 1), jnp.float32)],
)


def _tc_mid_body(*refs):
    zs = refs[:NQ]
    dinv, conv_W, conv_b = refs[NQ:NQ + 3]
    ys = refs[NQ + 3:]
    di = dinv[...]
    z = jnp.concatenate([zq[...] for zq in zs], axis=1)
    h = jnp.maximum(z * di + conv_b[...], 0.0)
    y = jnp.dot(h, conv_W[...], preferred_element_type=jnp.float32) * di
    _split(y, ys)


_tc_mid = pl.pallas_call(
    _tc_mid_body,
    grid=(GRID,),
    in_specs=[_qspec] * NQ + [
        _dspec,
        pl.BlockSpec((WIDTH, WIDTH), lambda i: (0, 0)),
        pl.BlockSpec((1, WIDTH), lambda i: (0, 0)),
    ],
    out_specs=[_qspec] * NQ,
    out_shape=[_qshape] * NQ,
)


def _tc_post_body(*refs):
    zs = refs[:NQ]
    dinv, conv_b, fc2_W, fc2_b, out = refs[NQ:]
    di = dinv[...]
    z = jnp.concatenate([zq[...] for zq in zs], axis=1)
    h = jnp.maximum(z * di + conv_b[...], 0.0)
    out[...] = jnp.dot(h, fc2_W[...], preferred_element_type=jnp.float32) + fc2_b[...]


_tc_post = pl.pallas_call(
    _tc_post_body,
    grid=(GRID,),
    in_specs=[_qspec] * NQ + [
        _dspec,
        pl.BlockSpec((1, WIDTH), lambda i: (0, 0)),
        pl.BlockSpec((WIDTH, 1), lambda i: (0, 0)),
        pl.BlockSpec((1, 1), lambda i: (0, 0)),
    ],
    out_specs=pl.BlockSpec((BN, 1), lambda i: (i, 0)),
    out_shape=jax.ShapeDtypeStruct((N_PAD, 1), jnp.float32),
)


def kernel(x, edge_index, fc1_W, fc1_b, conv_W, conv_b, fc2_W, fc2_b):
    src = jnp.concatenate(
        [edge_index[0], jnp.zeros((E_PAD - E,), jnp.int32)]).reshape(EROWS, LANES)
    dst = jnp.concatenate(
        [edge_index[1], jnp.full((E_PAD - E,), N, jnp.int32)]).reshape(EROWS, LANES)
    x_pad = jnp.concatenate([x, jnp.zeros((N_PAD - N, 3), x.dtype)], axis=0)

    sc_scatter = _sc_kernels()
    ones_q = jnp.ones((N_PAD, QW), jnp.float32)
    fc1_b2 = fc1_b.reshape(1, WIDTH)
    conv_b2 = conv_b.reshape(1, WIDTH)

    def body(carry, it):
        ys = carry[:NQ]
        dinv = carry[2 * NQ]
        z = sc_scatter(*ys, src, dst)

        def first(_):
            return _tc_pre(x_pad, z[0], fc1_W, fc1_b2, conv_W)

        def rest(_):
            ny = _tc_mid(*z, dinv, conv_W, conv_b2)
            return (*ny, dinv)

        nys = lax.cond(it == 0, first, rest, 0)
        return (*nys[:NQ], *z, nys[NQ]), None

    dinv0 = jnp.zeros((N_PAD, 1), jnp.float32)
    carry, _ = lax.scan(
        body,
        (*([ones_q] * NQ), *([ones_q] * NQ), dinv0),
        jnp.arange(DEPTH + 1), length=DEPTH + 1)
    zs = carry[NQ:2 * NQ]
    dinv = carry[2 * NQ]
    out = _tc_post(*zs, dinv, conv_b2, fc2_W, fc2_b.reshape(1, 1))
    return out[:N]


# speedup vs baseline: 4.9788x; 1.3825x over previous
import functools

import jax
import jax.numpy as jnp
from jax import lax
from jax.experimental import pallas as pl
from jax.experimental.pallas import tpu as pltpu
from jax.experimental.pallas import tpu_sc as plsc

N = 50000
E = 800000
WIDTH = 64
NQ = 8
QW = WIDTH // NQ
DEPTH = 4

LANES = 128
N_PAD = 50176
EROWS = 6400
E_PAD = EROWS * LANES
RPT = N_PAD // 16
ERPT = EROWS // 16


def _sc_scatter_body(*refs):
    ys = refs[:NQ]
    src2d, dst2d = refs[NQ], refs[NQ + 1]
    zs = refs[NQ + 2:2 * NQ + 2]
    srcv, dstv, rows_a, rows_b, z_sh, sem_a, sem_b = refs[2 * NQ + 2:]
    cid = lax.axis_index("c")
    sid = lax.axis_index("s")
    pltpu.sync_copy(src2d.at[pl.ds(sid * ERPT, ERPT)], srcv)
    pltpu.sync_copy(dst2d.at[pl.ds(sid * ERPT, ERPT)], dstv)

    def one_slice(y_hbm, z_hbm):
        pltpu.sync_copy(y_hbm.at[pl.ds(sid * RPT, RPT)],
                        z_sh.at[pl.ds(sid * RPT, RPT)])
        plsc.subcore_barrier()

        pltpu.async_copy(y_hbm.at[srcv.at[0]], rows_a, sem_a)

        def pairstep(k, carry):
            a = 2 * k
            pltpu.make_async_copy(y_hbm.at[srcv.at[a]], rows_a, sem_a).wait()
            pltpu.async_copy(y_hbm.at[srcv.at[a + 1]], rows_b, sem_b)
            pltpu.sync_copy(rows_a, z_sh.at[dstv.at[a]], add=True)

            @pl.when(a + 2 < ERPT)
            def _():
                pltpu.async_copy(y_hbm.at[srcv.at[a + 2]], rows_a, sem_a)

            pltpu.make_async_copy(y_hbm.at[srcv.at[a + 1]], rows_b, sem_b).wait()
            pltpu.sync_copy(rows_b, z_sh.at[dstv.at[a + 1]], add=True)
            return carry

        lax.fori_loop(0, ERPT // 2, pairstep, 0)
        plsc.subcore_barrier()
        pltpu.sync_copy(z_sh.at[pl.ds(sid * RPT, RPT)],
                        z_hbm.at[pl.ds(sid * RPT, RPT)])

    def core(c):
        for q in range(c, NQ, 2):
            one_slice(ys[q], zs[q])

    pl.when(cid == 0)(lambda: core(0))
    pl.when(cid == 1)(lambda: core(1))


@functools.cache
def _sc_kernels():
    mesh = plsc.VectorSubcoreMesh(core_axis_name="c", subcore_axis_name="s")
    params = pltpu.CompilerParams(use_tc_tiling_on_sc=False)
    qshape = jax.ShapeDtypeStruct((N_PAD, QW), jnp.float32)
    scatter = pl.kernel(
        _sc_scatter_body,
        out_type=[qshape] * NQ,
        mesh=mesh,
        scratch_types=[
            pltpu.VMEM((ERPT, LANES), jnp.int32),
            pltpu.VMEM((ERPT, LANES), jnp.int32),
            pltpu.VMEM((LANES, QW), jnp.float32),
            pltpu.VMEM((LANES, QW), jnp.float32),
            pltpu.VMEM_SHARED((N_PAD, QW), jnp.float32),
            pltpu.SemaphoreType.DMA,
            pltpu.SemaphoreType.DMA,
        ],
        compiler_params=params,
    )
    return scatter


BN = 1024
GRID = N_PAD // BN


def _split(y, outs):
    for q, ref in enumerate(outs):
        ref[...] = y[:, q * QW:(q + 1) * QW]


def _tc_pre_body(x, degz, fc1_W, fc1_b, conv_W, *outs):
    di = lax.rsqrt(degz[:, 0:1])
    h = jnp.dot(x[...], fc1_W[...], preferred_element_type=jnp.float32)
    h = h + fc1_b[...]
    y = jnp.dot(h, conv_W[...], preferred_element_type=jnp.float32) * di
    _split(y, outs[:NQ])
    outs[NQ][...] = di


_qspec = pl.BlockSpec((BN, QW), lambda i: (i, 0))
_qshape = jax.ShapeDtypeStruct((N_PAD, QW), jnp.float32)
_dspec = pl.BlockSpec((BN, 1), lambda i: (i, 0))

_tc_pre = pl.pallas_call(
    _tc_pre_body,
    grid=(GRID,),
    in_specs=[
        pl.BlockSpec((BN, 3), lambda i: (i, 0)),
        _qspec,
        pl.BlockSpec((3, WIDTH), lambda i: (0, 0)),
        pl.BlockSpec((1, WIDTH), lambda i: (0, 0)),
        pl.BlockSpec((WIDTH, WIDTH), lambda i: (0, 0)),
    ],
    out_specs=[_qspec] * NQ + [_dspec],
    out_shape=[_qshape] * NQ + [jax.ShapeDtypeStruct((N_PAD, 1), jnp.float32)],
)


def _tc_mid_body(*refs):
    zs = refs[:NQ]
    dinv, conv_W, conv_b = refs[NQ:NQ + 3]
    ys = refs[NQ + 3:]
    di = dinv[...]
    z = jnp.concatenate([zq[...] for zq in zs], axis=1)
    h = jnp.maximum(z * di + conv_b[...], 0.0)
    y = jnp.dot(h, conv_W[...], preferred_element_type=jnp.float32) * di
    _split(y, ys)


_tc_mid = pl.pallas_call(
    _tc_mid_body,
    grid=(GRID,),
    in_specs=[_qspec] * NQ + [
        _dspec,
        pl.BlockSpec((WIDTH, WIDTH), lambda i: (0, 0)),
        pl.BlockSpec((1, WIDTH), lambda i: (0, 0)),
    ],
    out_specs=[_qspec] * NQ,
    out_shape=[_qshape] * NQ,
)


def _tc_post_body(*refs):
    zs = refs[:NQ]
    dinv, conv_b, fc2_W, fc2_b, out = refs[NQ:]
    di = dinv[...]
    z = jnp.concatenate([zq[...] for zq in zs], axis=1)
    h = jnp.maximum(z * di + conv_b[...], 0.0)
    out[...] = jnp.dot(h, fc2_W[...], preferred_element_type=jnp.float32) + fc2_b[...]


_tc_post = pl.pallas_call(
    _tc_post_body,
    grid=(GRID,),
    in_specs=[_qspec] * NQ + [
        _dspec,
        pl.BlockSpec((1, WIDTH), lambda i: (0, 0)),
        pl.BlockSpec((WIDTH, 1), lambda i: (0, 0)),
        pl.BlockSpec((1, 1), lambda i: (0, 0)),
    ],
    out_specs=pl.BlockSpec((BN, 1), lambda i: (i, 0)),
    out_shape=jax.ShapeDtypeStruct((N_PAD, 1), jnp.float32),
)


def kernel(x, edge_index, fc1_W, fc1_b, conv_W, conv_b, fc2_W, fc2_b):
    src = jnp.concatenate(
        [edge_index[0], jnp.zeros((E_PAD - E,), jnp.int32)]).reshape(EROWS, LANES)
    dst = jnp.concatenate(
        [edge_index[1], jnp.full((E_PAD - E,), N, jnp.int32)]).reshape(EROWS, LANES)
    x_pad = jnp.concatenate([x, jnp.zeros((N_PAD - N, 3), x.dtype)], axis=0)

    sc_scatter = _sc_kernels()
    ones_q = jnp.ones((N_PAD, QW), jnp.float32)
    fc1_b2 = fc1_b.reshape(1, WIDTH)
    conv_b2 = conv_b.reshape(1, WIDTH)

    def body(carry, it):
        ys = carry[:NQ]
        dinv = carry[2 * NQ]
        z = sc_scatter(*ys, src, dst)

        def first(_):
            return _tc_pre(x_pad, z[0], fc1_W, fc1_b2, conv_W)

        def rest(_):
            ny = _tc_mid(*z, dinv, conv_W, conv_b2)
            return (*ny, dinv)

        nys = lax.cond(it == 0, first, rest, 0)
        return (*nys[:NQ], *z, nys[NQ]), None

    dinv0 = jnp.zeros((N_PAD, 1), jnp.float32)
    carry, _ = lax.scan(
        body,
        (*([ones_q] * NQ), *([ones_q] * NQ), dinv0),
        jnp.arange(DEPTH + 1), length=DEPTH + 1)
    zs = carry[NQ:2 * NQ]
    dinv = carry[2 * NQ]
    out = _tc_post(*zs, dinv, conv_b2, fc2_W, fc2_b.reshape(1, 1))
    return out[:N]
